# baseline (speedup 1.0000x reference)
import jax
import jax.numpy as jnp
from jax.experimental import pallas as pl
from jax.experimental.pallas import tpu as pltpu

EPS = 1e-10
_BLK = 512


def _body(pred_ref, tar_ref, mask_ref, out_ref, s_acc, c_acc):
    i = pl.program_id(0)

    @pl.when(i == 0)
    def _():
        s_acc[0] = 0.0
        c_acc[0] = 0.0

    m = mask_ref[...].astype(jnp.float32)
    t = jnp.log(tar_ref[...] + EPS)
    s = jnp.sum(jnp.abs(pred_ref[...] - t) * m)
    s_acc[0] += s
    c_acc[0] += jnp.sum(m)

    @pl.when(i == pl.num_programs(0) - 1)
    def _():
        out_ref[0] = s_acc[0] / (c_acc[0] * tar_ref.shape[-1])


def kernel(log_predicted, linear_tar, stft_length_masks):
    B, T, F = log_predicted.shape
    N = B * T
    pred = log_predicted.reshape(N, F)
    tar = linear_tar.reshape(N, F)
    mask = stft_length_masks.reshape(N, 1)

    out = pl.pallas_call(
        _body,
        grid=(N // _BLK,),
        in_specs=[
            pl.BlockSpec((_BLK, F), lambda i: (i, 0)),
            pl.BlockSpec((_BLK, F), lambda i: (i, 0)),
            pl.BlockSpec((_BLK, 1), lambda i: (i, 0)),
        ],
        out_specs=pl.BlockSpec(memory_space=pltpu.SMEM),
        out_shape=jax.ShapeDtypeStruct((1,), jnp.float32),
        scratch_shapes=[
            pltpu.SMEM((1,), jnp.float32),
            pltpu.SMEM((1,), jnp.float32),
        ],
    )(pred, tar, mask)
    return out[0]


# speedup vs baseline: 1.0804x; 1.0804x over previous
import jax
import jax.numpy as jnp
from jax.experimental import pallas as pl
from jax.experimental.pallas import tpu as pltpu

EPS = 1e-10
_BLK = 512


def _body(pred_ref, tar_ref, mask_ref, out_ref, s_acc, c_acc):
    i = pl.program_id(0)
    F = tar_ref.shape[-1]

    @pl.when(i == 0)
    def _():
        s_acc[...] = jnp.zeros_like(s_acc)
        c_acc[...] = jnp.zeros_like(c_acc)

    m = mask_ref[...].astype(jnp.float32)
    t = jnp.log(tar_ref[...] + EPS)
    d = jnp.abs(pred_ref[...] - t) * m
    s_acc[...] += jnp.sum(d.reshape(_BLK // 8, 8, F), axis=0)
    c_acc[...] += jnp.sum(m.reshape(_BLK // 8, 8, 1), axis=0)

    @pl.when(i == pl.num_programs(0) - 1)
    def _():
        out_ref[...] = (jnp.sum(s_acc[...]) / (jnp.sum(c_acc[...]) * F)).reshape(1, 1)


def kernel(log_predicted, linear_tar, stft_length_masks):
    B, T, F = log_predicted.shape
    N = B * T
    pred = log_predicted.reshape(N, F)
    tar = linear_tar.reshape(N, F)
    mask = stft_length_masks.reshape(N, 1)

    out = pl.pallas_call(
        _body,
        grid=(N // _BLK,),
        in_specs=[
            pl.BlockSpec((_BLK, F), lambda i: (i, 0)),
            pl.BlockSpec((_BLK, F), lambda i: (i, 0)),
            pl.BlockSpec((_BLK, 1), lambda i: (i, 0)),
        ],
        out_specs=pl.BlockSpec((1, 1), lambda i: (0, 0)),
        out_shape=jax.ShapeDtypeStruct((1, 1), jnp.float32),
        scratch_shapes=[
            pltpu.VMEM((8, F), jnp.float32),
            pltpu.VMEM((8, 1), jnp.float32),
        ],
    )(pred, tar, mask)
    return out[0, 0]
